# baseline (speedup 1.0000x reference)
import functools

import jax
import jax.numpy as jnp
from jax.experimental import pallas as pl
from jax.experimental.pallas import tpu as pltpu


def _softmax_kernel(g_ref, w_ref):
    g = g_ref[...].astype(jnp.float32)
    m = jnp.max(g, axis=-1, keepdims=True)
    e = jnp.exp(g - m)
    w_ref[...] = e * pl.reciprocal(jnp.sum(e, axis=-1, keepdims=True),
                                   approx=False)


def _pool_kernel(w_ref, x_ref, o_ref, *, tb):
    w = w_ref[...]
    for b in range(tb):
        xb = x_ref[b]
        o_ref[0, b, :] = jnp.sum(xb * w, axis=0)


def kernel(x, gsp):
    B, C, H, W = x.shape
    HW = H * W
    assert gsp.shape == (C, HW)

    vmem_limit = 64 * 1024 * 1024

    w = pl.pallas_call(
        _softmax_kernel,
        out_shape=jax.ShapeDtypeStruct((C, HW), jnp.float32),
        in_specs=[pl.BlockSpec((C, HW), lambda: (0, 0))],
        out_specs=pl.BlockSpec((C, HW), lambda: (0, 0)),
        compiler_params=pltpu.CompilerParams(vmem_limit_bytes=vmem_limit),
        cost_estimate=pl.CostEstimate(
            flops=4 * C * HW,
            transcendentals=C * HW,
            bytes_accessed=2 * C * HW * 4),
    )(gsp)
    wT = w.T.reshape(HW, C)

    xT = jnp.transpose(x, (0, 2, 3, 1)).reshape(B, HW, C)

    TB = 4
    while B % TB != 0:
        TB //= 2
    nb = B // TB

    pool_fn = functools.partial(_pool_kernel, tb=TB)

    x_bytes = B * C * HW * 4
    out = pl.pallas_call(
        pool_fn,
        out_shape=jax.ShapeDtypeStruct((nb, TB, C), x.dtype),
        grid=(nb,),
        in_specs=[
            pl.BlockSpec((HW, C), lambda b: (0, 0)),
            pl.BlockSpec((TB, HW, C), lambda b: (b, 0, 0)),
        ],
        out_specs=pl.BlockSpec((1, TB, C), lambda b: (b, 0, 0)),
        compiler_params=pltpu.CompilerParams(
            dimension_semantics=("parallel",),
            vmem_limit_bytes=vmem_limit),
        cost_estimate=pl.CostEstimate(
            flops=2 * B * C * HW,
            transcendentals=0,
            bytes_accessed=x_bytes + C * HW * 4 + B * C * 4),
    )(wT, xT)
    return out.reshape(B, C)


# speedup vs baseline: 1.0259x; 1.0259x over previous
import functools

import jax
import jax.numpy as jnp
from jax.experimental import pallas as pl
from jax.experimental.pallas import tpu as pltpu


def _softmax_kernel(g_ref, w_ref):
    g = g_ref[...].astype(jnp.float32)
    m = jnp.max(g, axis=-1, keepdims=True)
    e = jnp.exp(g - m)
    w_ref[...] = e * pl.reciprocal(jnp.sum(e, axis=-1, keepdims=True),
                                   approx=False)


def _pool_kernel(w_ref, x_ref, o_ref, *, tb):
    w = w_ref[...]
    for b in range(tb):
        xb = x_ref[b]
        o_ref[0, b, :] = jnp.sum(xb * w, axis=0)


def kernel(x, gsp):
    B, C, H, W = x.shape
    HW = H * W
    assert gsp.shape == (C, HW)

    vmem_limit = 64 * 1024 * 1024

    w = pl.pallas_call(
        _softmax_kernel,
        out_shape=jax.ShapeDtypeStruct((C, HW), jnp.float32),
        in_specs=[pl.BlockSpec((C, HW), lambda: (0, 0))],
        out_specs=pl.BlockSpec((C, HW), lambda: (0, 0)),
        compiler_params=pltpu.CompilerParams(vmem_limit_bytes=vmem_limit),
        cost_estimate=pl.CostEstimate(
            flops=4 * C * HW,
            transcendentals=C * HW,
            bytes_accessed=2 * C * HW * 4),
    )(gsp)
    wT = w.T.reshape(HW, C)

    xT = jnp.transpose(x, (0, 2, 3, 1)).reshape(B, HW, C)

    TB = 8
    while B % TB != 0:
        TB //= 2
    nb = B // TB

    pool_fn = functools.partial(_pool_kernel, tb=TB)

    x_bytes = B * C * HW * 4
    out = pl.pallas_call(
        pool_fn,
        out_shape=jax.ShapeDtypeStruct((nb, TB, C), x.dtype),
        grid=(nb,),
        in_specs=[
            pl.BlockSpec((HW, C), lambda b: (0, 0)),
            pl.BlockSpec((TB, HW, C), lambda b: (b, 0, 0)),
        ],
        out_specs=pl.BlockSpec((1, TB, C), lambda b: (b, 0, 0)),
        compiler_params=pltpu.CompilerParams(
            dimension_semantics=("parallel",),
            vmem_limit_bytes=vmem_limit),
        cost_estimate=pl.CostEstimate(
            flops=2 * B * C * HW,
            transcendentals=0,
            bytes_accessed=x_bytes + C * HW * 4 + B * C * 4),
    )(wT, xT)
    return out.reshape(B, C)
